# Initial kernel scaffold; baseline (speedup 1.0000x reference)
import functools

import jax
import jax.numpy as jnp
from jax import lax
from jax.experimental import pallas as pl
from jax.experimental.pallas import tpu as pltpu
from jax.experimental.pallas import tpu_sc as plsc

NC = 2
NS = 16
_F32 = jnp.float32


def _sc_mesh():
    return plsc.VectorSubcoreMesh(
        core_axis_name="c", subcore_axis_name="s", num_cores=NC,
        num_subcores=NS)


def _deg_tables(dst, n):
    e = dst.shape[0]
    assert e % (NC * 128) == 0
    chunks_per_sc = e // (NC * 128)
    jmax = pl.cdiv(chunks_per_sc, NS)
    rows_per_worker = n // NS
    assert n % NS == 0 and rows_per_worker % 125 == 0
    zsteps = rows_per_worker // 125

    @functools.partial(
        pl.kernel,
        out_type=(jax.ShapeDtypeStruct((n, 16), _F32),
                  jax.ShapeDtypeStruct((n, 16), _F32)),
        mesh=_sc_mesh(),
        scratch_types=[
            pltpu.VMEM((128, 16), _F32),
            pltpu.VMEM((125, 16), _F32),
            pltpu.VMEM((128,), jnp.int32),
            pltpu.VMEM_SHARED((n, 16), _F32),
        ],
    )
    def k(dst_hbm, t0_hbm, t1_hbm, onehot, zbuf, idx, table):
        c = lax.axis_index("c")
        s = lax.axis_index("s")
        lane = lax.iota(jnp.int32, 16)
        one_row = jnp.where(lane == 0, 1.0, 0.0).astype(_F32)
        zero_row = jnp.zeros((16,), _F32)

        @pl.loop(0, 128)
        def _(i):
            onehot[i, :] = one_row

        @pl.loop(0, 125)
        def _(i):
            zbuf[i, :] = zero_row

        @pl.loop(0, zsteps)
        def _(j):
            pltpu.sync_copy(
                zbuf, table.at[pl.ds(s * rows_per_worker + j * 125, 125)])

        plsc.subcore_barrier()

        @pl.loop(0, jmax)
        def _(j):
            g = j * NS + s

            @pl.when(g < chunks_per_sc)
            def _():
                base = (c * chunks_per_sc + g) * 128
                pltpu.sync_copy(dst_hbm.at[pl.ds(base, 128)], idx)
                pltpu.sync_copy(onehot, table.at[idx], add=True)

        plsc.subcore_barrier()

        row0 = s * rows_per_worker
        sl = pl.ds(row0, rows_per_worker)

        @pl.when(c == 0)
        def _():
            pltpu.sync_copy(table.at[sl], t0_hbm.at[sl])

        @pl.when(c == 1)
        def _():
            pltpu.sync_copy(table.at[sl], t1_hbm.at[sl])

    return k(dst)


def _aggregate(xs_a, xs_b, src, dst):
    n, fh = xs_a.shape
    e = src.shape[0]
    per_worker = e // NS
    assert e % NS == 0 and per_worker % 16 == 0
    full = per_worker // 128
    tail = per_worker - full * 128
    rows_per_worker = n // NS
    assert n % NS == 0

    out_t = jax.ShapeDtypeStruct((n, fh), _F32)

    @functools.partial(
        pl.kernel,
        out_type=(out_t, out_t),
        mesh=_sc_mesh(),
        scratch_types=[
            pltpu.VMEM((128,), jnp.int32),
            pltpu.VMEM((128,), jnp.int32),
            pltpu.VMEM((tail,), jnp.int32),
            pltpu.VMEM((tail,), jnp.int32),
            pltpu.VMEM((128, fh), _F32),
            pltpu.VMEM_SHARED((n, fh), _F32),
        ],
    )
    def k(xsa_hbm, xsb_hbm, src_hbm, dst_hbm, oa_hbm, ob_hbm,
          idxs, idxd, idxs_t, idxd_t, rows, acc):
        c = lax.axis_index("c")
        s = lax.axis_index("s")
        row_sl = pl.ds(s * rows_per_worker, rows_per_worker)
        ebase = s * per_worker

        def run(xs_hbm, out_hbm):
            pltpu.sync_copy(xs_hbm.at[row_sl], acc.at[row_sl])
            plsc.subcore_barrier()

            @pl.loop(0, full)
            def _(i):
                b = ebase + i * 128
                pltpu.sync_copy(src_hbm.at[pl.ds(b, 128)], idxs)
                pltpu.sync_copy(xs_hbm.at[idxs], rows)
                pltpu.sync_copy(dst_hbm.at[pl.ds(b, 128)], idxd)
                pltpu.sync_copy(rows, acc.at[idxd], add=True)

            if tail:
                b = ebase + full * 128
                pltpu.sync_copy(src_hbm.at[pl.ds(b, tail)], idxs_t)
                pltpu.sync_copy(xs_hbm.at[idxs_t], rows.at[pl.ds(0, tail)])
                pltpu.sync_copy(dst_hbm.at[pl.ds(b, tail)], idxd_t)
                pltpu.sync_copy(rows.at[pl.ds(0, tail)], acc.at[idxd_t],
                                add=True)

            plsc.subcore_barrier()
            pltpu.sync_copy(acc.at[row_sl], out_hbm.at[row_sl])

        @pl.when(c == 0)
        def _():
            run(xsa_hbm, oa_hbm)

        @pl.when(c == 1)
        def _():
            run(xsb_hbm, ob_hbm)

    return k(xs_a, xs_b, src, dst)


_BR = 2000


def _tc_a(t0, t1, x_a, x_b):
    n = t0.shape[0]
    grid = n // _BR

    def body(t0b, t1b, xab, xbb, dvb, oa, ob):
        deg = t0b[:, 0:1] + t1b[:, 0:1] + 1.0
        dinv = lax.rsqrt(deg)
        dvb[...] = jnp.broadcast_to(dinv, (_BR, 16))
        oa[...] = xab[...] * dinv
        ob[...] = xbb[...] * dinv

    rb = lambda f: pl.BlockSpec((_BR, f), lambda i: (i, 0))
    return pl.pallas_call(
        body,
        grid=(grid,),
        in_specs=[rb(16), rb(16), rb(32), rb(32)],
        out_specs=[rb(16), rb(32), rb(32)],
        out_shape=(jax.ShapeDtypeStruct((n, 16), _F32),
                   jax.ShapeDtypeStruct((n, 32), _F32),
                   jax.ShapeDtypeStruct((n, 32), _F32)),
    )(t0, t1, x_a, x_b)


def _tc_b(acc_a, acc_b, dinvr, w1a, w1b, b1, w2a, w2b):
    n = acc_a.shape[0]
    grid = n // _BR
    dot = functools.partial(jnp.dot, preferred_element_type=_F32,
                            precision=lax.Precision.HIGHEST)

    def body(aab, abb, dvb, w1ab, w1bb, b1b, w2ab, w2bb, oa, ob):
        dinv = dvb[:, 0:1]
        ha = aab[...] * dinv
        hb = abb[...] * dinv
        z = jnp.maximum(dot(ha, w1ab[...]) + dot(hb, w1bb[...]) + b1b[...],
                        0.0)
        oa[...] = dot(z, w2ab[...]) * dinv
        ob[...] = dot(z, w2bb[...]) * dinv

    rb = lambda f: pl.BlockSpec((_BR, f), lambda i: (i, 0))
    fullb = lambda a: pl.BlockSpec(a.shape, lambda i: (0, 0))
    return pl.pallas_call(
        body,
        grid=(grid,),
        in_specs=[rb(32), rb(32), rb(16), fullb(w1a), fullb(w1b), fullb(b1),
                  fullb(w2a), fullb(w2b)],
        out_specs=[rb(32), rb(32)],
        out_shape=(jax.ShapeDtypeStruct((n, 32), _F32),
                   jax.ShapeDtypeStruct((n, 32), _F32)),
    )(acc_a, acc_b, dinvr, w1a, w1b, b1, w2a, w2b)


def _tc_c(acc_a, acc_b, dinvr, b2a, b2b):
    n = acc_a.shape[0]
    grid = n // _BR

    def body(aab, abb, dvb, b2ab, b2bb, oa, ob):
        dinv = dvb[:, 0:1]
        oa[...] = jnp.maximum(aab[...] * dinv + b2ab[...], 0.0)
        ob[...] = jnp.maximum(abb[...] * dinv + b2bb[...], 0.0)

    rb = lambda f: pl.BlockSpec((_BR, f), lambda i: (i, 0))
    fullb = lambda a: pl.BlockSpec(a.shape, lambda i: (0, 0))
    return pl.pallas_call(
        body,
        grid=(grid,),
        in_specs=[rb(32), rb(32), rb(16), fullb(b2a), fullb(b2b)],
        out_specs=[rb(32), rb(32)],
        out_shape=(jax.ShapeDtypeStruct((n, 32), _F32),
                   jax.ShapeDtypeStruct((n, 32), _F32)),
    )(acc_a, acc_b, dinvr, b2a, b2b)


def kernel(x, edge_index, W1, b1, W2, b2):
    n, f_in = x.shape
    src = edge_index[0]
    dst = edge_index[1]
    fh = f_in // 2

    x_a = x[:, :fh]
    x_b = x[:, fh:]
    w1a = W1[:fh]
    w1b = W1[fh:]
    f2 = W2.shape[1]
    w2a = W2[:, : f2 // 2]
    w2b = W2[:, f2 // 2:]
    b1r = b1.reshape(1, -1)
    b2a = b2[: f2 // 2].reshape(1, -1)
    b2b = b2[f2 // 2:].reshape(1, -1)

    t0, t1 = _deg_tables(dst, n)
    dinvr, xs_a, xs_b = _tc_a(t0, t1, x_a, x_b)
    acc1a, acc1b = _aggregate(xs_a, xs_b, src, dst)
    xs2a, xs2b = _tc_b(acc1a, acc1b, dinvr, w1a, w1b, b1r, w2a, w2b)
    acc2a, acc2b = _aggregate(xs2a, xs2b, src, dst)
    out_a, out_b = _tc_c(acc2a, acc2b, dinvr, b2a, b2b)
    return jnp.concatenate([out_a, out_b], axis=1)


# speedup vs baseline: 11.3388x; 11.3388x over previous
import functools

import jax
import jax.numpy as jnp
from jax import lax
from jax.experimental import pallas as pl
from jax.experimental.pallas import tpu as pltpu
from jax.experimental.pallas import tpu_sc as plsc

NC = 2
NS = 16
_F32 = jnp.float32


def _sc_mesh():
    return plsc.VectorSubcoreMesh(
        core_axis_name="c", subcore_axis_name="s", num_cores=NC,
        num_subcores=NS)


_CP = pltpu.CompilerParams(use_tc_tiling_on_sc=False)


def _interleaved(total, s, fn):
    jmax = pl.cdiv(total, NS)

    @pl.loop(0, jmax)
    def _(j):
        g = j * NS + s

        @pl.when(g < total)
        def _():
            fn(g)


def _deg_tables(dst, n):
    e = dst.shape[0]
    assert e % (NC * 128) == 0
    chunks_per_sc = e // (NC * 128)
    zc = 2000
    assert n % zc == 0

    @functools.partial(
        pl.kernel,
        out_type=(jax.ShapeDtypeStruct((n,), _F32),
                  jax.ShapeDtypeStruct((n,), _F32)),
        mesh=_sc_mesh(), compiler_params=_CP,
        scratch_types=[
            pltpu.VMEM((zc,), _F32),
            pltpu.VMEM((128,), _F32),
            pltpu.VMEM((128,), jnp.int32),
            pltpu.VMEM_SHARED((n,), _F32),
        ],
    )
    def k(dst_hbm, d0_hbm, d1_hbm, zbuf, ones, idx, table):
        c = lax.axis_index("c")
        s = lax.axis_index("s")

        @pl.loop(0, zc // 16)
        def _(i):
            zbuf[pl.ds(i * 16, 16)] = jnp.zeros((16,), _F32)

        @pl.loop(0, 8)
        def _(i):
            ones[pl.ds(i * 16, 16)] = jnp.full((16,), 1.0, _F32)

        _interleaved(n // zc, s, lambda g: pltpu.sync_copy(
            zbuf, table.at[pl.ds(g * zc, zc)]))
        plsc.subcore_barrier()

        def edge_chunk(g):
            base = (c * chunks_per_sc + g) * 128
            pltpu.sync_copy(dst_hbm.at[pl.ds(base, 128)], idx)
            pltpu.sync_copy(ones, table.at[idx], add=True)

        _interleaved(chunks_per_sc, s, edge_chunk)
        plsc.subcore_barrier()

        @pl.when(c == 0)
        def _():
            _interleaved(n // zc, s, lambda g: pltpu.sync_copy(
                table.at[pl.ds(g * zc, zc)], d0_hbm.at[pl.ds(g * zc, zc)]))

        @pl.when(c == 1)
        def _():
            _interleaved(n // zc, s, lambda g: pltpu.sync_copy(
                table.at[pl.ds(g * zc, zc)], d1_hbm.at[pl.ds(g * zc, zc)]))

    return k(dst)


def _aggregate(xs_a, xs_b, src, dst):
    n, fh = xs_a.shape
    e = src.shape[0]
    per_worker = e // NS
    assert e % NS == 0 and per_worker % 16 == 0
    full = per_worker // 128
    tail = per_worker - full * 128
    rowc = 200
    assert n % rowc == 0

    out_t = jax.ShapeDtypeStruct((n, fh), _F32)

    @functools.partial(
        pl.kernel,
        out_type=(out_t, out_t),
        mesh=_sc_mesh(), compiler_params=_CP,
        scratch_types=[
            pltpu.VMEM((128,), jnp.int32),
            pltpu.VMEM((128,), jnp.int32),
            pltpu.VMEM((tail,), jnp.int32),
            pltpu.VMEM((tail,), jnp.int32),
            pltpu.VMEM((128, fh), _F32),
            pltpu.VMEM_SHARED((n, fh), _F32),
        ],
    )
    def k(xsa_hbm, xsb_hbm, src_hbm, dst_hbm, oa_hbm, ob_hbm,
          idxs, idxd, idxs_t, idxd_t, rows, acc):
        c = lax.axis_index("c")
        s = lax.axis_index("s")
        ebase = s * per_worker

        def run(xs_hbm, out_hbm):
            _interleaved(n // rowc, s, lambda g: pltpu.sync_copy(
                xs_hbm.at[pl.ds(g * rowc, rowc)],
                acc.at[pl.ds(g * rowc, rowc)]))
            plsc.subcore_barrier()

            @pl.loop(0, full)
            def _(i):
                b = ebase + i * 128
                pltpu.sync_copy(src_hbm.at[pl.ds(b, 128)], idxs)
                pltpu.sync_copy(xs_hbm.at[idxs], rows)
                pltpu.sync_copy(dst_hbm.at[pl.ds(b, 128)], idxd)
                pltpu.sync_copy(rows, acc.at[idxd], add=True)

            if tail:
                b = ebase + full * 128
                pltpu.sync_copy(src_hbm.at[pl.ds(b, tail)], idxs_t)
                pltpu.sync_copy(xs_hbm.at[idxs_t], rows.at[pl.ds(0, tail)])
                pltpu.sync_copy(dst_hbm.at[pl.ds(b, tail)], idxd_t)
                pltpu.sync_copy(rows.at[pl.ds(0, tail)], acc.at[idxd_t],
                                add=True)

            plsc.subcore_barrier()
            _interleaved(n // rowc, s, lambda g: pltpu.sync_copy(
                acc.at[pl.ds(g * rowc, rowc)],
                out_hbm.at[pl.ds(g * rowc, rowc)]))

        @pl.when(c == 0)
        def _():
            run(xsa_hbm, oa_hbm)

        @pl.when(c == 1)
        def _():
            run(xsb_hbm, ob_hbm)

    return k(xs_a, xs_b, src, dst)


_BR = 2000


def _rb(f):
    return pl.BlockSpec((_BR, f), lambda i: (i, 0))


def _fullb(a):
    return pl.BlockSpec(a.shape, lambda i: (0,) * a.ndim)


def _tc_a(d0, d1, x_a, x_b):
    n = d0.shape[0]
    grid = n // _BR

    def body(d0b, d1b, xab, xbb, dvb, oa, ob):
        deg = d0b[...] + d1b[...] + 1.0
        dinv = lax.rsqrt(deg)
        dvb[...] = dinv
        oa[...] = xab[...] * dinv
        ob[...] = xbb[...] * dinv

    return pl.pallas_call(
        body,
        grid=(grid,),
        in_specs=[_rb(1), _rb(1), _rb(32), _rb(32)],
        out_specs=[_rb(1), _rb(32), _rb(32)],
        out_shape=(jax.ShapeDtypeStruct((n, 1), _F32),
                   jax.ShapeDtypeStruct((n, 32), _F32),
                   jax.ShapeDtypeStruct((n, 32), _F32)),
    )(d0, d1, x_a, x_b)


def _tc_b(acc_a, acc_b, dinv, w1a, w1b, b1, w2a, w2b):
    n = acc_a.shape[0]
    grid = n // _BR
    dot = functools.partial(jnp.dot, preferred_element_type=_F32,
                            precision=lax.Precision.HIGHEST)

    def body(aab, abb, dvb, w1ab, w1bb, b1b, w2ab, w2bb, oa, ob):
        dinv = dvb[...]
        z = (dot(aab[...] * dinv, w1ab[...])
             + dot(abb[...] * dinv, w1bb[...]) + b1b[...])
        z = jnp.maximum(z, 0.0)
        oa[...] = dot(z, w2ab[...]) * dinv
        ob[...] = dot(z, w2bb[...]) * dinv

    return pl.pallas_call(
        body,
        grid=(grid,),
        in_specs=[_rb(32), _rb(32), _rb(1), _fullb(w1a), _fullb(w1b),
                  _fullb(b1), _fullb(w2a), _fullb(w2b)],
        out_specs=[_rb(32), _rb(32)],
        out_shape=(jax.ShapeDtypeStruct((n, 32), _F32),
                   jax.ShapeDtypeStruct((n, 32), _F32)),
    )(acc_a, acc_b, dinv, w1a, w1b, b1, w2a, w2b)


def _tc_c(acc_a, acc_b, dinv, b2a, b2b):
    n = acc_a.shape[0]
    grid = n // _BR

    def body(aab, abb, dvb, b2ab, b2bb, oa, ob):
        dinv = dvb[...]
        oa[...] = jnp.maximum(aab[...] * dinv + b2ab[...], 0.0)
        ob[...] = jnp.maximum(abb[...] * dinv + b2bb[...], 0.0)

    return pl.pallas_call(
        body,
        grid=(grid,),
        in_specs=[_rb(32), _rb(32), _rb(1), _fullb(b2a), _fullb(b2b)],
        out_specs=[_rb(32), _rb(32)],
        out_shape=(jax.ShapeDtypeStruct((n, 32), _F32),
                   jax.ShapeDtypeStruct((n, 32), _F32)),
    )(acc_a, acc_b, dinv, b2a, b2b)


def kernel(x, edge_index, W1, b1, W2, b2):
    n, f_in = x.shape
    src = edge_index[0]
    dst = edge_index[1]
    fh = f_in // 2
    f2 = W2.shape[1]

    x_a = x[:, :fh]
    x_b = x[:, fh:]
    w1a = W1[:fh]
    w1b = W1[fh:]
    w2a = W2[:, : f2 // 2]
    w2b = W2[:, f2 // 2:]
    b1r = b1.reshape(1, -1)
    b2a = b2[: f2 // 2].reshape(1, -1)
    b2b = b2[f2 // 2:].reshape(1, -1)

    d0, d1 = _deg_tables(dst, n)
    dinv, xs_a, xs_b = _tc_a(d0.reshape(n, 1), d1.reshape(n, 1), x_a, x_b)
    acc1a, acc1b = _aggregate(xs_a, xs_b, src, dst)
    xs2a, xs2b = _tc_b(acc1a, acc1b, dinv, w1a, w1b, b1r, w2a, w2b)
    acc2a, acc2b = _aggregate(xs2a, xs2b, src, dst)
    out_a, out_b = _tc_c(acc2a, acc2b, dinv, b2a, b2b)
    return jnp.concatenate([out_a, out_b], axis=1)


# speedup vs baseline: 19.5119x; 1.7208x over previous
import functools

import jax
import jax.numpy as jnp
from jax import lax
from jax.experimental import pallas as pl
from jax.experimental.pallas import tpu as pltpu
from jax.experimental.pallas import tpu_sc as plsc

NC = 2
NS = 16
_F32 = jnp.float32


def _sc_mesh():
    return plsc.VectorSubcoreMesh(
        core_axis_name="c", subcore_axis_name="s", num_cores=NC,
        num_subcores=NS)


_CP = pltpu.CompilerParams(use_tc_tiling_on_sc=False)


def _interleaved(total, s, fn):
    jmax = pl.cdiv(total, NS)

    @pl.loop(0, jmax)
    def _(j):
        g = j * NS + s

        @pl.when(g < total)
        def _():
            fn(g)


def _deg_tables(dst, n):
    e = dst.shape[0]
    assert e % (NC * NS) == 0
    per_worker = e // (NC * NS)
    full = per_worker // 128
    tail = per_worker - full * 128
    pipe = (full // 2) * 2
    zc = 2000
    assert n % zc == 0 and per_worker % 8 == 0

    @functools.partial(
        pl.kernel,
        out_type=(jax.ShapeDtypeStruct((n,), _F32),
                  jax.ShapeDtypeStruct((n,), _F32)),
        mesh=_sc_mesh(), compiler_params=_CP,
        scratch_types=[
            pltpu.VMEM((zc,), _F32),
            pltpu.VMEM((128,), _F32),
            pltpu.VMEM((2, 128), jnp.int32),
            pltpu.VMEM((tail,), jnp.int32),
            pltpu.VMEM_SHARED((n,), _F32),
            pltpu.SemaphoreType.DMA((2,)),
            pltpu.SemaphoreType.DMA((2,)),
        ],
    )
    def k(dst_hbm, d0_hbm, d1_hbm, zbuf, ones, idx2, idxt, table,
          sem_i, sem_s):
        c = lax.axis_index("c")
        s = lax.axis_index("s")

        @pl.loop(0, zc // 16)
        def _(i):
            zbuf[pl.ds(i * 16, 16)] = jnp.zeros((16,), _F32)

        @pl.loop(0, 8)
        def _(i):
            ones[pl.ds(i * 16, 16)] = jnp.full((16,), 1.0, _F32)

        _interleaved(n // zc, s, lambda g: pltpu.sync_copy(
            zbuf, table.at[pl.ds(g * zc, zc)]))
        plsc.subcore_barrier()

        ebase = (c * NS + s) * per_worker

        @pl.loop(0, pipe // 2)
        def _(ii):
            for b in range(2):
                i = ii * 2 + b

                @pl.when(ii > 0)
                def _():
                    pltpu.make_async_copy(
                        ones, table.at[idx2.at[b]], sem_s.at[b]).wait()

                pltpu.async_copy(dst_hbm.at[pl.ds(ebase + i * 128, 128)],
                                 idx2.at[b], sem_i.at[b]).wait()
                pltpu.async_copy(ones, table.at[idx2.at[b]], sem_s.at[b],
                                 add=True)

        for b in range(2):
            pltpu.make_async_copy(
                ones, table.at[idx2.at[b]], sem_s.at[b]).wait()

        @pl.loop(0, full - pipe)
        def _(i):
            base = ebase + (pipe + i) * 128
            pltpu.sync_copy(dst_hbm.at[pl.ds(base, 128)], idx2.at[0])
            pltpu.sync_copy(ones, table.at[idx2.at[0]], add=True)

        if tail:
            base = ebase + full * 128
            pltpu.sync_copy(dst_hbm.at[pl.ds(base, tail)], idxt)
            pltpu.sync_copy(ones.at[pl.ds(0, tail)], table.at[idxt],
                            add=True)

        plsc.subcore_barrier()

        @pl.when(c == 0)
        def _():
            _interleaved(n // zc, s, lambda g: pltpu.sync_copy(
                table.at[pl.ds(g * zc, zc)], d0_hbm.at[pl.ds(g * zc, zc)]))

        @pl.when(c == 1)
        def _():
            _interleaved(n // zc, s, lambda g: pltpu.sync_copy(
                table.at[pl.ds(g * zc, zc)], d1_hbm.at[pl.ds(g * zc, zc)]))

    return k(dst)


def _aggregate(xs_a, xs_b, src, dst):
    n, fh = xs_a.shape
    e = src.shape[0]
    per_worker = e // NS
    assert e % NS == 0 and per_worker % 16 == 0
    full = per_worker // 128
    tail = per_worker - full * 128
    rowc = 200
    assert n % rowc == 0

    out_t = jax.ShapeDtypeStruct((n, fh), _F32)

    pipe = (full // 2) * 2

    @functools.partial(
        pl.kernel,
        out_type=(out_t, out_t),
        mesh=_sc_mesh(), compiler_params=_CP,
        scratch_types=[
            pltpu.VMEM((2, 128), jnp.int32),
            pltpu.VMEM((2, 128), jnp.int32),
            pltpu.VMEM((tail,), jnp.int32),
            pltpu.VMEM((tail,), jnp.int32),
            pltpu.VMEM((2, 128, fh), _F32),
            pltpu.VMEM_SHARED((n, fh), _F32),
            pltpu.SemaphoreType.DMA((2,)),
            pltpu.SemaphoreType.DMA((2,)),
            pltpu.SemaphoreType.DMA((2,)),
            pltpu.SemaphoreType.DMA((2,)),
        ],
    )
    def k(xsa_hbm, xsb_hbm, src_hbm, dst_hbm, oa_hbm, ob_hbm,
          idxs2, idxd2, idxs_t, idxd_t, rows2, acc,
          sem_is, sem_id, sem_g, sem_s):
        c = lax.axis_index("c")
        s = lax.axis_index("s")
        ebase = s * per_worker

        def run(xs_hbm, out_hbm):
            _interleaved(n // rowc, s, lambda g: pltpu.sync_copy(
                xs_hbm.at[pl.ds(g * rowc, rowc)],
                acc.at[pl.ds(g * rowc, rowc)]))
            plsc.subcore_barrier()

            @pl.loop(0, pipe // 2)
            def _(ii):
                for b in range(2):
                    i = ii * 2 + b
                    nb = 1 - b

                    @pl.when(ii > 0)
                    def _():
                        pltpu.make_async_copy(
                            rows2.at[b], acc.at[idxd2.at[b]],
                            sem_s.at[b]).wait()

                    dis = pltpu.async_copy(
                        src_hbm.at[pl.ds(ebase + i * 128, 128)],
                        idxs2.at[b], sem_is.at[b])
                    pltpu.async_copy(
                        dst_hbm.at[pl.ds(ebase + i * 128, 128)],
                        idxd2.at[b], sem_id.at[b])
                    dis.wait()
                    pltpu.async_copy(xs_hbm.at[idxs2.at[b]], rows2.at[b],
                                     sem_g.at[b])

                    @pl.when(i >= 1)
                    def _():
                        pltpu.make_async_copy(
                            xs_hbm.at[idxs2.at[nb]], rows2.at[nb],
                            sem_g.at[nb]).wait()
                        pltpu.make_async_copy(
                            dst_hbm.at[pl.ds(0, 128)], idxd2.at[nb],
                            sem_id.at[nb]).wait()
                        pltpu.async_copy(rows2.at[nb],
                                         acc.at[idxd2.at[nb]],
                                         sem_s.at[nb], add=True)

            lb = (pipe - 1) % 2
            pltpu.make_async_copy(xs_hbm.at[idxs2.at[lb]], rows2.at[lb],
                                  sem_g.at[lb]).wait()
            pltpu.make_async_copy(dst_hbm.at[pl.ds(0, 128)],
                                  idxd2.at[lb], sem_id.at[lb]).wait()
            pltpu.make_async_copy(rows2.at[1 - lb],
                                  acc.at[idxd2.at[1 - lb]],
                                  sem_s.at[1 - lb]).wait()
            pltpu.sync_copy(rows2.at[lb], acc.at[idxd2.at[lb]], add=True)

            if tail:
                b0 = ebase + full * 128
                pltpu.sync_copy(src_hbm.at[pl.ds(b0, tail)], idxs_t)
                pltpu.sync_copy(xs_hbm.at[idxs_t],
                                rows2.at[0].at[pl.ds(0, tail)])
                pltpu.sync_copy(dst_hbm.at[pl.ds(b0, tail)], idxd_t)
                pltpu.sync_copy(rows2.at[0].at[pl.ds(0, tail)],
                                acc.at[idxd_t], add=True)

            plsc.subcore_barrier()
            _interleaved(n // rowc, s, lambda g: pltpu.sync_copy(
                acc.at[pl.ds(g * rowc, rowc)],
                out_hbm.at[pl.ds(g * rowc, rowc)]))
            plsc.subcore_barrier()

        @pl.when(c == 0)
        def _():
            run(xsa_hbm, oa_hbm)

        @pl.when(c == 1)
        def _():
            run(xsb_hbm, ob_hbm)

    return k(xs_a, xs_b, src, dst)


_BR = 2000


def _rb(f):
    return pl.BlockSpec((_BR, f), lambda i: (i, 0))


def _fullb(a):
    return pl.BlockSpec(a.shape, lambda i: (0,) * a.ndim)


def _tc_a(d0, d1, x_a, x_b):
    n = d0.shape[0]
    grid = n // _BR

    def body(d0b, d1b, xab, xbb, dvb, oa, ob):
        deg = d0b[...] + d1b[...] + 1.0
        dinv = lax.rsqrt(deg)
        dvb[...] = dinv
        oa[...] = xab[...] * dinv
        ob[...] = xbb[...] * dinv

    return pl.pallas_call(
        body,
        grid=(grid,),
        in_specs=[_rb(1), _rb(1), _rb(32), _rb(32)],
        out_specs=[_rb(1), _rb(32), _rb(32)],
        out_shape=(jax.ShapeDtypeStruct((n, 1), _F32),
                   jax.ShapeDtypeStruct((n, 32), _F32),
                   jax.ShapeDtypeStruct((n, 32), _F32)),
    )(d0, d1, x_a, x_b)


def _tc_b(acc_a, acc_b, dinv, w1a, w1b, b1, w2a, w2b):
    n = acc_a.shape[0]
    grid = n // _BR
    dot = functools.partial(jnp.dot, preferred_element_type=_F32,
                            precision=lax.Precision.HIGHEST)

    def body(aab, abb, dvb, w1ab, w1bb, b1b, w2ab, w2bb, oa, ob):
        dinv = dvb[...]
        z = (dot(aab[...] * dinv, w1ab[...])
             + dot(abb[...] * dinv, w1bb[...]) + b1b[...])
        z = jnp.maximum(z, 0.0)
        oa[...] = dot(z, w2ab[...]) * dinv
        ob[...] = dot(z, w2bb[...]) * dinv

    return pl.pallas_call(
        body,
        grid=(grid,),
        in_specs=[_rb(32), _rb(32), _rb(1), _fullb(w1a), _fullb(w1b),
                  _fullb(b1), _fullb(w2a), _fullb(w2b)],
        out_specs=[_rb(32), _rb(32)],
        out_shape=(jax.ShapeDtypeStruct((n, 32), _F32),
                   jax.ShapeDtypeStruct((n, 32), _F32)),
    )(acc_a, acc_b, dinv, w1a, w1b, b1, w2a, w2b)


def _tc_c(acc_a, acc_b, dinv, b2a, b2b):
    n = acc_a.shape[0]
    grid = n // _BR

    def body(aab, abb, dvb, b2ab, b2bb, oa, ob):
        dinv = dvb[...]
        oa[...] = jnp.maximum(aab[...] * dinv + b2ab[...], 0.0)
        ob[...] = jnp.maximum(abb[...] * dinv + b2bb[...], 0.0)

    return pl.pallas_call(
        body,
        grid=(grid,),
        in_specs=[_rb(32), _rb(32), _rb(1), _fullb(b2a), _fullb(b2b)],
        out_specs=[_rb(32), _rb(32)],
        out_shape=(jax.ShapeDtypeStruct((n, 32), _F32),
                   jax.ShapeDtypeStruct((n, 32), _F32)),
    )(acc_a, acc_b, dinv, b2a, b2b)


def kernel(x, edge_index, W1, b1, W2, b2):
    n, f_in = x.shape
    src = edge_index[0]
    dst = edge_index[1]
    fh = f_in // 2
    f2 = W2.shape[1]

    x_a = x[:, :fh]
    x_b = x[:, fh:]
    w1a = W1[:fh]
    w1b = W1[fh:]
    w2a = W2[:, : f2 // 2]
    w2b = W2[:, f2 // 2:]
    b1r = b1.reshape(1, -1)
    b2a = b2[: f2 // 2].reshape(1, -1)
    b2b = b2[f2 // 2:].reshape(1, -1)

    d0, d1 = _deg_tables(dst, n)
    dinv, xs_a, xs_b = _tc_a(d0.reshape(n, 1), d1.reshape(n, 1), x_a, x_b)
    acc1a, acc1b = _aggregate(xs_a, xs_b, src, dst)
    xs2a, xs2b = _tc_b(acc1a, acc1b, dinv, w1a, w1b, b1r, w2a, w2b)
    acc2a, acc2b = _aggregate(xs2a, xs2b, src, dst)
    out_a, out_b = _tc_c(acc2a, acc2b, dinv, b2a, b2b)
    return jnp.concatenate([out_a, out_b], axis=1)


# speedup vs baseline: 22.5592x; 1.1562x over previous
import functools

import jax
import jax.numpy as jnp
from jax import lax
from jax.experimental import pallas as pl
from jax.experimental.pallas import tpu as pltpu
from jax.experimental.pallas import tpu_sc as plsc

NC = 2
NS = 16
_F32 = jnp.float32


def _sc_mesh():
    return plsc.VectorSubcoreMesh(
        core_axis_name="c", subcore_axis_name="s", num_cores=NC,
        num_subcores=NS)


_CP = pltpu.CompilerParams(use_tc_tiling_on_sc=False)


def _interleaved(total, s, fn):
    jmax = pl.cdiv(total, NS)

    @pl.loop(0, jmax)
    def _(j):
        g = j * NS + s

        @pl.when(g < total)
        def _():
            fn(g)


def _deg_tables(dst, n):
    e = dst.shape[0]
    assert e % (NC * NS) == 0
    per_worker = e // (NC * NS)
    full = per_worker // 128
    tail = per_worker - full * 128
    pipe = (full // 2) * 2
    zc = 2000
    assert n % zc == 0 and per_worker % 8 == 0

    @functools.partial(
        pl.kernel,
        out_type=(jax.ShapeDtypeStruct((n,), _F32),
                  jax.ShapeDtypeStruct((n,), _F32)),
        mesh=_sc_mesh(), compiler_params=_CP,
        scratch_types=[
            pltpu.VMEM((zc,), _F32),
            pltpu.VMEM((128,), _F32),
            pltpu.VMEM((2, 128), jnp.int32),
            pltpu.VMEM((tail,), jnp.int32),
            pltpu.VMEM_SHARED((n,), _F32),
            pltpu.SemaphoreType.DMA((2,)),
            pltpu.SemaphoreType.DMA((2,)),
        ],
    )
    def k(dst_hbm, d0_hbm, d1_hbm, zbuf, ones, idx2, idxt, table,
          sem_i, sem_s):
        c = lax.axis_index("c")
        s = lax.axis_index("s")

        @pl.loop(0, zc // 16)
        def _(i):
            zbuf[pl.ds(i * 16, 16)] = jnp.zeros((16,), _F32)

        @pl.loop(0, 8)
        def _(i):
            ones[pl.ds(i * 16, 16)] = jnp.full((16,), 1.0, _F32)

        _interleaved(n // zc, s, lambda g: pltpu.sync_copy(
            zbuf, table.at[pl.ds(g * zc, zc)]))
        plsc.subcore_barrier()

        ebase = (c * NS + s) * per_worker

        @pl.loop(0, pipe // 2)
        def _(ii):
            for b in range(2):
                i = ii * 2 + b

                @pl.when(ii > 0)
                def _():
                    pltpu.make_async_copy(
                        ones, table.at[idx2.at[b]], sem_s.at[b]).wait()

                pltpu.async_copy(dst_hbm.at[pl.ds(ebase + i * 128, 128)],
                                 idx2.at[b], sem_i.at[b]).wait()
                pltpu.async_copy(ones, table.at[idx2.at[b]], sem_s.at[b],
                                 add=True)

        for b in range(2):
            pltpu.make_async_copy(
                ones, table.at[idx2.at[b]], sem_s.at[b]).wait()

        @pl.loop(0, full - pipe)
        def _(i):
            base = ebase + (pipe + i) * 128
            pltpu.sync_copy(dst_hbm.at[pl.ds(base, 128)], idx2.at[0])
            pltpu.sync_copy(ones, table.at[idx2.at[0]], add=True)

        if tail:
            base = ebase + full * 128
            pltpu.sync_copy(dst_hbm.at[pl.ds(base, tail)], idxt)
            pltpu.sync_copy(ones.at[pl.ds(0, tail)], table.at[idxt],
                            add=True)

        plsc.subcore_barrier()

        @pl.when(c == 0)
        def _():
            _interleaved(n // zc, s, lambda g: pltpu.sync_copy(
                table.at[pl.ds(g * zc, zc)], d0_hbm.at[pl.ds(g * zc, zc)]))

        @pl.when(c == 1)
        def _():
            _interleaved(n // zc, s, lambda g: pltpu.sync_copy(
                table.at[pl.ds(g * zc, zc)], d1_hbm.at[pl.ds(g * zc, zc)]))

    return k(dst)


def _aggregate(xs_a, xs_b, src, dst):
    n, fh = xs_a.shape
    e = src.shape[0]
    per_worker = e // NS
    assert e % NS == 0 and per_worker % 16 == 0
    full = per_worker // 128
    tail = per_worker - full * 128
    rowc = 200
    assert n % rowc == 0

    out_t = jax.ShapeDtypeStruct((n, fh), _F32)

    pipe = (full // 4) * 4

    @functools.partial(
        pl.kernel,
        out_type=(out_t, out_t),
        mesh=_sc_mesh(), compiler_params=_CP,
        scratch_types=[
            pltpu.VMEM((4, 128), jnp.int32),
            pltpu.VMEM((4, 128), jnp.int32),
            pltpu.VMEM((tail,), jnp.int32),
            pltpu.VMEM((tail,), jnp.int32),
            pltpu.VMEM((2, 128, fh), _F32),
            pltpu.VMEM_SHARED((n, fh), _F32),
            pltpu.SemaphoreType.DMA((4,)),
            pltpu.SemaphoreType.DMA((4,)),
            pltpu.SemaphoreType.DMA((2,)),
            pltpu.SemaphoreType.DMA((2,)),
        ],
    )
    def k(xsa_hbm, xsb_hbm, src_hbm, dst_hbm, oa_hbm, ob_hbm,
          idxs4, idxd4, idxs_t, idxd_t, rows2, acc,
          sem_is, sem_id, sem_g, sem_s):
        c = lax.axis_index("c")
        s = lax.axis_index("s")
        ebase = s * per_worker

        def idx_load(i, q):
            pltpu.async_copy(src_hbm.at[pl.ds(ebase + i * 128, 128)],
                             idxs4.at[q], sem_is.at[q])
            pltpu.async_copy(dst_hbm.at[pl.ds(ebase + i * 128, 128)],
                             idxd4.at[q], sem_id.at[q])

        def run(xs_hbm, out_hbm):
            _interleaved(n // rowc, s, lambda g: pltpu.sync_copy(
                xs_hbm.at[pl.ds(g * rowc, rowc)],
                acc.at[pl.ds(g * rowc, rowc)]))
            plsc.subcore_barrier()

            idx_load(0, 0)
            idx_load(1, 1)

            @pl.loop(0, pipe // 4)
            def _(ii):
                for kk in range(4):
                    b = kk % 2
                    nb = 1 - b
                    q = kk
                    qp = (kk + 2) % 4
                    qn = (kk + 3) % 4
                    i = ii * 4 + kk

                    @pl.when(i >= 2)
                    def _():
                        pltpu.make_async_copy(
                            rows2.at[b], acc.at[idxd4.at[qp]],
                            sem_s.at[b]).wait()

                    @pl.when(i + 2 < pipe)
                    def _():
                        idx_load(i + 2, qp)

                    pltpu.make_async_copy(
                        src_hbm.at[pl.ds(0, 128)], idxs4.at[q],
                        sem_is.at[q]).wait()
                    pltpu.async_copy(xs_hbm.at[idxs4.at[q]], rows2.at[b],
                                     sem_g.at[b])

                    @pl.when(i >= 1)
                    def _():
                        pltpu.make_async_copy(
                            xs_hbm.at[idxs4.at[qn]], rows2.at[nb],
                            sem_g.at[nb]).wait()
                        pltpu.make_async_copy(
                            dst_hbm.at[pl.ds(0, 128)], idxd4.at[qn],
                            sem_id.at[qn]).wait()
                        pltpu.async_copy(rows2.at[nb],
                                         acc.at[idxd4.at[qn]],
                                         sem_s.at[nb], add=True)

            lb = (pipe - 1) % 2
            lq = (pipe - 1) % 4
            pltpu.make_async_copy(xs_hbm.at[idxs4.at[lq]], rows2.at[lb],
                                  sem_g.at[lb]).wait()
            pltpu.make_async_copy(dst_hbm.at[pl.ds(0, 128)],
                                  idxd4.at[lq], sem_id.at[lq]).wait()
            pltpu.make_async_copy(rows2.at[1 - lb],
                                  acc.at[idxd4.at[(pipe - 2) % 4]],
                                  sem_s.at[1 - lb]).wait()
            pltpu.sync_copy(rows2.at[lb], acc.at[idxd4.at[lq]], add=True)

            @pl.loop(0, full - pipe)
            def _(i):
                b0 = ebase + (pipe + i) * 128
                pltpu.sync_copy(src_hbm.at[pl.ds(b0, 128)], idxs4.at[0])
                pltpu.sync_copy(xs_hbm.at[idxs4.at[0]], rows2.at[0])
                pltpu.sync_copy(dst_hbm.at[pl.ds(b0, 128)], idxd4.at[0])
                pltpu.sync_copy(rows2.at[0], acc.at[idxd4.at[0]], add=True)

            if tail:
                b0 = ebase + full * 128
                pltpu.sync_copy(src_hbm.at[pl.ds(b0, tail)], idxs_t)
                pltpu.sync_copy(xs_hbm.at[idxs_t],
                                rows2.at[0].at[pl.ds(0, tail)])
                pltpu.sync_copy(dst_hbm.at[pl.ds(b0, tail)], idxd_t)
                pltpu.sync_copy(rows2.at[0].at[pl.ds(0, tail)],
                                acc.at[idxd_t], add=True)

            plsc.subcore_barrier()
            _interleaved(n // rowc, s, lambda g: pltpu.sync_copy(
                acc.at[pl.ds(g * rowc, rowc)],
                out_hbm.at[pl.ds(g * rowc, rowc)]))
            plsc.subcore_barrier()

        @pl.when(c == 0)
        def _():
            run(xsa_hbm, oa_hbm)

        @pl.when(c == 1)
        def _():
            run(xsb_hbm, ob_hbm)

    return k(xs_a, xs_b, src, dst)


_BR = 2000


def _rb(f):
    return pl.BlockSpec((_BR, f), lambda i: (i, 0))


def _fullb(a):
    return pl.BlockSpec(a.shape, lambda i: (0,) * a.ndim)


def _tc_a(d0, d1, x_a, x_b):
    n = d0.shape[0]
    grid = n // _BR

    def body(d0b, d1b, xab, xbb, dvb, oa, ob):
        deg = d0b[...] + d1b[...] + 1.0
        dinv = lax.rsqrt(deg)
        dvb[...] = dinv
        oa[...] = xab[...] * dinv
        ob[...] = xbb[...] * dinv

    return pl.pallas_call(
        body,
        grid=(grid,),
        in_specs=[_rb(1), _rb(1), _rb(32), _rb(32)],
        out_specs=[_rb(1), _rb(32), _rb(32)],
        out_shape=(jax.ShapeDtypeStruct((n, 1), _F32),
                   jax.ShapeDtypeStruct((n, 32), _F32),
                   jax.ShapeDtypeStruct((n, 32), _F32)),
    )(d0, d1, x_a, x_b)


def _tc_b(acc_a, acc_b, dinv, w1a, w1b, b1, w2a, w2b):
    n = acc_a.shape[0]
    grid = n // _BR
    dot = functools.partial(jnp.dot, preferred_element_type=_F32,
                            precision=lax.Precision.HIGHEST)

    def body(aab, abb, dvb, w1ab, w1bb, b1b, w2ab, w2bb, oa, ob):
        dinv = dvb[...]
        z = (dot(aab[...] * dinv, w1ab[...])
             + dot(abb[...] * dinv, w1bb[...]) + b1b[...])
        z = jnp.maximum(z, 0.0)
        oa[...] = dot(z, w2ab[...]) * dinv
        ob[...] = dot(z, w2bb[...]) * dinv

    return pl.pallas_call(
        body,
        grid=(grid,),
        in_specs=[_rb(32), _rb(32), _rb(1), _fullb(w1a), _fullb(w1b),
                  _fullb(b1), _fullb(w2a), _fullb(w2b)],
        out_specs=[_rb(32), _rb(32)],
        out_shape=(jax.ShapeDtypeStruct((n, 32), _F32),
                   jax.ShapeDtypeStruct((n, 32), _F32)),
    )(acc_a, acc_b, dinv, w1a, w1b, b1, w2a, w2b)


def _tc_c(acc_a, acc_b, dinv, b2a, b2b):
    n = acc_a.shape[0]
    grid = n // _BR

    def body(aab, abb, dvb, b2ab, b2bb, oa, ob):
        dinv = dvb[...]
        oa[...] = jnp.maximum(aab[...] * dinv + b2ab[...], 0.0)
        ob[...] = jnp.maximum(abb[...] * dinv + b2bb[...], 0.0)

    return pl.pallas_call(
        body,
        grid=(grid,),
        in_specs=[_rb(32), _rb(32), _rb(1), _fullb(b2a), _fullb(b2b)],
        out_specs=[_rb(32), _rb(32)],
        out_shape=(jax.ShapeDtypeStruct((n, 32), _F32),
                   jax.ShapeDtypeStruct((n, 32), _F32)),
    )(acc_a, acc_b, dinv, b2a, b2b)


def kernel(x, edge_index, W1, b1, W2, b2):
    n, f_in = x.shape
    src = edge_index[0]
    dst = edge_index[1]
    fh = f_in // 2
    f2 = W2.shape[1]

    x_a = x[:, :fh]
    x_b = x[:, fh:]
    w1a = W1[:fh]
    w1b = W1[fh:]
    w2a = W2[:, : f2 // 2]
    w2b = W2[:, f2 // 2:]
    b1r = b1.reshape(1, -1)
    b2a = b2[: f2 // 2].reshape(1, -1)
    b2b = b2[f2 // 2:].reshape(1, -1)

    d0, d1 = _deg_tables(dst, n)
    dinv, xs_a, xs_b = _tc_a(d0.reshape(n, 1), d1.reshape(n, 1), x_a, x_b)
    acc1a, acc1b = _aggregate(xs_a, xs_b, src, dst)
    xs2a, xs2b = _tc_b(acc1a, acc1b, dinv, w1a, w1b, b1r, w2a, w2b)
    acc2a, acc2b = _aggregate(xs2a, xs2b, src, dst)
    out_a, out_b = _tc_c(acc2a, acc2b, dinv, b2a, b2b)
    return jnp.concatenate([out_a, out_b], axis=1)


# speedup vs baseline: 23.7477x; 1.0527x over previous
import functools

import jax
import jax.numpy as jnp
from jax import lax
from jax.experimental import pallas as pl
from jax.experimental.pallas import tpu as pltpu
from jax.experimental.pallas import tpu_sc as plsc

NC = 2
NS = 16
_F32 = jnp.float32


def _sc_mesh():
    return plsc.VectorSubcoreMesh(
        core_axis_name="c", subcore_axis_name="s", num_cores=NC,
        num_subcores=NS)


_CP = pltpu.CompilerParams(use_tc_tiling_on_sc=False)


def _interleaved(total, s, fn):
    jmax = pl.cdiv(total, NS)

    @pl.loop(0, jmax)
    def _(j):
        g = j * NS + s

        @pl.when(g < total)
        def _():
            fn(g)


_CH = 128


def _deg_tables(dst, n):
    e = dst.shape[0]
    assert e % (NC * NS) == 0
    per_worker = e // (NC * NS)
    full = per_worker // _CH
    tail = per_worker - full * _CH
    pipe = (full // 4) * 4
    zc = 2000
    assert n % zc == 0 and per_worker % 8 == 0

    @functools.partial(
        pl.kernel,
        out_type=(jax.ShapeDtypeStruct((n,), _F32),
                  jax.ShapeDtypeStruct((n,), _F32)),
        mesh=_sc_mesh(), compiler_params=_CP,
        scratch_types=[
            pltpu.VMEM((zc,), _F32),
            pltpu.VMEM((_CH,), _F32),
            pltpu.VMEM((4, _CH), jnp.int32),
            pltpu.VMEM((tail,), jnp.int32),
            pltpu.VMEM_SHARED((n,), _F32),
            pltpu.SemaphoreType.DMA((4,)),
            pltpu.SemaphoreType.DMA((2,)),
        ],
    )
    def k(dst_hbm, d0_hbm, d1_hbm, zbuf, ones, idx4, idxt, table,
          sem_i, sem_s):
        c = lax.axis_index("c")
        s = lax.axis_index("s")

        @pl.loop(0, zc // 16)
        def _(i):
            zbuf[pl.ds(i * 16, 16)] = jnp.zeros((16,), _F32)

        @pl.loop(0, _CH // 16)
        def _(i):
            ones[pl.ds(i * 16, 16)] = jnp.full((16,), 1.0, _F32)

        _interleaved(n // zc, s, lambda g: pltpu.sync_copy(
            zbuf, table.at[pl.ds(g * zc, zc)]))
        plsc.subcore_barrier()

        ebase = (c * NS + s) * per_worker

        def idx_load(i, q):
            pltpu.async_copy(dst_hbm.at[pl.ds(ebase + i * _CH, _CH)],
                             idx4.at[q], sem_i.at[q])

        idx_load(0, 0)
        idx_load(1, 1)

        @pl.loop(0, pipe // 4)
        def _(ii):
            for kk in range(4):
                b = kk % 2
                q = kk
                qp = (kk + 2) % 4
                i = ii * 4 + kk

                @pl.when(i >= 2)
                def _():
                    pltpu.make_async_copy(
                        ones, table.at[idx4.at[qp]], sem_s.at[b]).wait()

                @pl.when(i + 2 < pipe)
                def _():
                    idx_load(i + 2, qp)

                pltpu.make_async_copy(
                    dst_hbm.at[pl.ds(0, _CH)], idx4.at[q],
                    sem_i.at[q]).wait()
                pltpu.async_copy(ones, table.at[idx4.at[q]], sem_s.at[b],
                                 add=True)

        for b in range(2):
            pltpu.make_async_copy(
                ones, table.at[idx4.at[(pipe - 2 + b) % 4]],
                sem_s.at[(pipe - 2 + b) % 2]).wait()

        @pl.loop(0, full - pipe)
        def _(i):
            base = ebase + (pipe + i) * _CH
            pltpu.sync_copy(dst_hbm.at[pl.ds(base, _CH)], idx4.at[0])
            pltpu.sync_copy(ones, table.at[idx4.at[0]], add=True)

        if tail:
            base = ebase + full * _CH
            pltpu.sync_copy(dst_hbm.at[pl.ds(base, tail)], idxt)
            pltpu.sync_copy(ones.at[pl.ds(0, tail)], table.at[idxt],
                            add=True)

        plsc.subcore_barrier()

        @pl.when(c == 0)
        def _():
            _interleaved(n // zc, s, lambda g: pltpu.sync_copy(
                table.at[pl.ds(g * zc, zc)], d0_hbm.at[pl.ds(g * zc, zc)]))

        @pl.when(c == 1)
        def _():
            _interleaved(n // zc, s, lambda g: pltpu.sync_copy(
                table.at[pl.ds(g * zc, zc)], d1_hbm.at[pl.ds(g * zc, zc)]))

    return k(dst)


def _aggregate(xs_a, xs_b, src, dst):
    n, fh = xs_a.shape
    e = src.shape[0]
    per_worker = e // NS
    assert e % NS == 0 and per_worker % 16 == 0
    full = per_worker // _CH
    tail = per_worker - full * _CH
    rowc = 200
    assert n % rowc == 0

    out_t = jax.ShapeDtypeStruct((n, fh), _F32)

    pipe = (full // 4) * 4

    @functools.partial(
        pl.kernel,
        out_type=(out_t, out_t),
        mesh=_sc_mesh(), compiler_params=_CP,
        scratch_types=[
            pltpu.VMEM((4, _CH), jnp.int32),
            pltpu.VMEM((4, _CH), jnp.int32),
            pltpu.VMEM((tail,), jnp.int32),
            pltpu.VMEM((tail,), jnp.int32),
            pltpu.VMEM((2, _CH, fh), _F32),
            pltpu.VMEM_SHARED((n, fh), _F32),
            pltpu.SemaphoreType.DMA((4,)),
            pltpu.SemaphoreType.DMA((4,)),
            pltpu.SemaphoreType.DMA((2,)),
            pltpu.SemaphoreType.DMA((2,)),
        ],
    )
    def k(xsa_hbm, xsb_hbm, src_hbm, dst_hbm, oa_hbm, ob_hbm,
          idxs4, idxd4, idxs_t, idxd_t, rows2, acc,
          sem_is, sem_id, sem_g, sem_s):
        c = lax.axis_index("c")
        s = lax.axis_index("s")
        ebase = s * per_worker

        def idx_load(i, q):
            pltpu.async_copy(src_hbm.at[pl.ds(ebase + i * _CH, _CH)],
                             idxs4.at[q], sem_is.at[q])
            pltpu.async_copy(dst_hbm.at[pl.ds(ebase + i * _CH, _CH)],
                             idxd4.at[q], sem_id.at[q])

        def run(xs_hbm, out_hbm):
            _interleaved(n // rowc, s, lambda g: pltpu.sync_copy(
                xs_hbm.at[pl.ds(g * rowc, rowc)],
                acc.at[pl.ds(g * rowc, rowc)]))
            plsc.subcore_barrier()

            idx_load(0, 0)
            idx_load(1, 1)

            @pl.loop(0, pipe // 4)
            def _(ii):
                for kk in range(4):
                    b = kk % 2
                    nb = 1 - b
                    q = kk
                    qp = (kk + 2) % 4
                    qn = (kk + 3) % 4
                    i = ii * 4 + kk

                    @pl.when(i >= 2)
                    def _():
                        pltpu.make_async_copy(
                            rows2.at[b], acc.at[idxd4.at[qp]],
                            sem_s.at[b]).wait()

                    @pl.when(i + 2 < pipe)
                    def _():
                        idx_load(i + 2, qp)

                    pltpu.make_async_copy(
                        src_hbm.at[pl.ds(0, _CH)], idxs4.at[q],
                        sem_is.at[q]).wait()
                    pltpu.async_copy(xs_hbm.at[idxs4.at[q]], rows2.at[b],
                                     sem_g.at[b])

                    @pl.when(i >= 1)
                    def _():
                        pltpu.make_async_copy(
                            xs_hbm.at[idxs4.at[qn]], rows2.at[nb],
                            sem_g.at[nb]).wait()
                        pltpu.make_async_copy(
                            dst_hbm.at[pl.ds(0, _CH)], idxd4.at[qn],
                            sem_id.at[qn]).wait()
                        pltpu.async_copy(rows2.at[nb],
                                         acc.at[idxd4.at[qn]],
                                         sem_s.at[nb], add=True)

            lb = (pipe - 1) % 2
            lq = (pipe - 1) % 4
            pltpu.make_async_copy(xs_hbm.at[idxs4.at[lq]], rows2.at[lb],
                                  sem_g.at[lb]).wait()
            pltpu.make_async_copy(dst_hbm.at[pl.ds(0, _CH)],
                                  idxd4.at[lq], sem_id.at[lq]).wait()
            pltpu.make_async_copy(rows2.at[1 - lb],
                                  acc.at[idxd4.at[(pipe - 2) % 4]],
                                  sem_s.at[1 - lb]).wait()
            pltpu.sync_copy(rows2.at[lb], acc.at[idxd4.at[lq]], add=True)

            @pl.loop(0, full - pipe)
            def _(i):
                b0 = ebase + (pipe + i) * _CH
                pltpu.sync_copy(src_hbm.at[pl.ds(b0, _CH)], idxs4.at[0])
                pltpu.sync_copy(xs_hbm.at[idxs4.at[0]], rows2.at[0])
                pltpu.sync_copy(dst_hbm.at[pl.ds(b0, _CH)], idxd4.at[0])
                pltpu.sync_copy(rows2.at[0], acc.at[idxd4.at[0]], add=True)

            if tail:
                b0 = ebase + full * _CH
                pltpu.sync_copy(src_hbm.at[pl.ds(b0, tail)], idxs_t)
                pltpu.sync_copy(xs_hbm.at[idxs_t],
                                rows2.at[0].at[pl.ds(0, tail)])
                pltpu.sync_copy(dst_hbm.at[pl.ds(b0, tail)], idxd_t)
                pltpu.sync_copy(rows2.at[0].at[pl.ds(0, tail)],
                                acc.at[idxd_t], add=True)

            plsc.subcore_barrier()
            _interleaved(n // rowc, s, lambda g: pltpu.sync_copy(
                acc.at[pl.ds(g * rowc, rowc)],
                out_hbm.at[pl.ds(g * rowc, rowc)]))
            plsc.subcore_barrier()

        @pl.when(c == 0)
        def _():
            run(xsa_hbm, oa_hbm)

        @pl.when(c == 1)
        def _():
            run(xsb_hbm, ob_hbm)

    return k(xs_a, xs_b, src, dst)


_BR = 2000


def _rb(f):
    return pl.BlockSpec((_BR, f), lambda i: (i, 0))


def _fullb(a):
    return pl.BlockSpec(a.shape, lambda i: (0,) * a.ndim)


def _tc_a(d0, d1, x_a, x_b):
    n = d0.shape[0]
    grid = n // _BR

    def body(d0b, d1b, xab, xbb, dvb, oa, ob):
        deg = d0b[...] + d1b[...] + 1.0
        dinv = lax.rsqrt(deg)
        dvb[...] = dinv
        oa[...] = xab[...] * dinv
        ob[...] = xbb[...] * dinv

    return pl.pallas_call(
        body,
        grid=(grid,),
        in_specs=[_rb(1), _rb(1), _rb(32), _rb(32)],
        out_specs=[_rb(1), _rb(32), _rb(32)],
        out_shape=(jax.ShapeDtypeStruct((n, 1), _F32),
                   jax.ShapeDtypeStruct((n, 32), _F32),
                   jax.ShapeDtypeStruct((n, 32), _F32)),
    )(d0, d1, x_a, x_b)


def _tc_b(acc_a, acc_b, dinv, w1a, w1b, b1, w2a, w2b):
    n = acc_a.shape[0]
    grid = n // _BR
    dot = functools.partial(jnp.dot, preferred_element_type=_F32,
                            precision=lax.Precision.HIGHEST)

    def body(aab, abb, dvb, w1ab, w1bb, b1b, w2ab, w2bb, oa, ob):
        dinv = dvb[...]
        z = (dot(aab[...] * dinv, w1ab[...])
             + dot(abb[...] * dinv, w1bb[...]) + b1b[...])
        z = jnp.maximum(z, 0.0)
        oa[...] = dot(z, w2ab[...]) * dinv
        ob[...] = dot(z, w2bb[...]) * dinv

    return pl.pallas_call(
        body,
        grid=(grid,),
        in_specs=[_rb(32), _rb(32), _rb(1), _fullb(w1a), _fullb(w1b),
                  _fullb(b1), _fullb(w2a), _fullb(w2b)],
        out_specs=[_rb(32), _rb(32)],
        out_shape=(jax.ShapeDtypeStruct((n, 32), _F32),
                   jax.ShapeDtypeStruct((n, 32), _F32)),
    )(acc_a, acc_b, dinv, w1a, w1b, b1, w2a, w2b)


def _tc_c(acc_a, acc_b, dinv, b2a, b2b):
    n = acc_a.shape[0]
    grid = n // _BR

    def body(aab, abb, dvb, b2ab, b2bb, oa, ob):
        dinv = dvb[...]
        oa[...] = jnp.maximum(aab[...] * dinv + b2ab[...], 0.0)
        ob[...] = jnp.maximum(abb[...] * dinv + b2bb[...], 0.0)

    return pl.pallas_call(
        body,
        grid=(grid,),
        in_specs=[_rb(32), _rb(32), _rb(1), _fullb(b2a), _fullb(b2b)],
        out_specs=[_rb(32), _rb(32)],
        out_shape=(jax.ShapeDtypeStruct((n, 32), _F32),
                   jax.ShapeDtypeStruct((n, 32), _F32)),
    )(acc_a, acc_b, dinv, b2a, b2b)


def kernel(x, edge_index, W1, b1, W2, b2):
    n, f_in = x.shape
    src = edge_index[0]
    dst = edge_index[1]
    fh = f_in // 2
    f2 = W2.shape[1]

    x_a = x[:, :fh]
    x_b = x[:, fh:]
    w1a = W1[:fh]
    w1b = W1[fh:]
    w2a = W2[:, : f2 // 2]
    w2b = W2[:, f2 // 2:]
    b1r = b1.reshape(1, -1)
    b2a = b2[: f2 // 2].reshape(1, -1)
    b2b = b2[f2 // 2:].reshape(1, -1)

    d0, d1 = _deg_tables(dst, n)
    dinv, xs_a, xs_b = _tc_a(d0.reshape(n, 1), d1.reshape(n, 1), x_a, x_b)
    acc1a, acc1b = _aggregate(xs_a, xs_b, src, dst)
    xs2a, xs2b = _tc_b(acc1a, acc1b, dinv, w1a, w1b, b1r, w2a, w2b)
    acc2a, acc2b = _aggregate(xs2a, xs2b, src, dst)
    out_a, out_b = _tc_c(acc2a, acc2b, dinv, b2a, b2b)
    return jnp.concatenate([out_a, out_b], axis=1)
